# baseline (speedup 1.0000x reference)
import functools

import jax
import jax.numpy as jnp
from jax.experimental import pallas as pl
from jax.experimental.pallas import tpu as pltpu

D = 768
FF = 3072
E = 8
NTOK = 2048
TM = 256
NT = NTOK // TM
FBLK = 1024
NF = FF // FBLK

_NEG = -1e30


def _routing_combine(x, Wg, bg):
    s = jnp.dot(x, Wg, preferred_element_type=jnp.float32) + bg
    col = jax.lax.broadcasted_iota(jnp.int32, s.shape, 1)
    m1 = jnp.max(s, axis=1, keepdims=True)
    a1 = jnp.min(jnp.where(s == m1, col, E), axis=1, keepdims=True)
    s2 = jnp.where(col == a1, _NEG, s)
    m2 = jnp.max(s2, axis=1, keepdims=True)
    a2 = jnp.min(jnp.where(s2 == m2, col, E), axis=1, keepdims=True)
    e2 = jnp.exp(m2 - m1)
    w1 = 1.0 / (1.0 + e2)
    w2 = e2 / (1.0 + e2)
    comb = jnp.where(col == a1, w1, 0.0) + jnp.where(col == a2, w2, 0.0)
    return comb


def _dense_body(x_ref, Wg_ref, bg_ref, W1_ref, b1_ref, W2_ref, b2_ref,
                out_ref, comb_ref):
    e = pl.program_id(0)
    f = pl.program_id(1)
    t = pl.program_id(2)
    x = x_ref[...]

    @pl.when(jnp.logical_and(e == 0, f == 0))
    def _():
        comb_ref[pl.ds(t * TM, TM), :] = _routing_combine(x, Wg_ref[...],
                                                          bg_ref[...])

    b1e = b1_ref[pl.ds(e, 1), :]
    h = jnp.maximum(
        jnp.dot(x, W1_ref[0], preferred_element_type=jnp.float32) + b1e,
        0.0)
    y = jnp.dot(h, W2_ref[0], preferred_element_type=jnp.float32)
    comb = comb_ref[pl.ds(t * TM, TM), :]
    col = jax.lax.broadcasted_iota(jnp.int32, comb.shape, 1)
    c = jnp.sum(jnp.where(col == e, comb, 0.0), axis=1, keepdims=True)
    contrib = c * y

    @pl.when(f == 0)
    def _():
        contrib_b = contrib + c * b2_ref[pl.ds(e, 1), :]

        @pl.when(e == 0)
        def _():
            out_ref[pl.ds(t * TM, TM), :] = contrib_b

        @pl.when(e != 0)
        def _():
            out_ref[pl.ds(t * TM, TM), :] += contrib_b

    @pl.when(f != 0)
    def _():
        out_ref[pl.ds(t * TM, TM), :] += contrib


def kernel(x, Wg, bg, W1, b1, W2, b2):
    out = pl.pallas_call(
        _dense_body,
        grid=(E, NF, NT),
        in_specs=[
            pl.BlockSpec((TM, D), lambda e, f, t: (t, 0)),
            pl.BlockSpec((D, E), lambda e, f, t: (0, 0)),
            pl.BlockSpec((1, E), lambda e, f, t: (0, 0)),
            pl.BlockSpec((1, D, FBLK), lambda e, f, t: (e, 0, f)),
            pl.BlockSpec((E, FBLK), lambda e, f, t: (0, f)),
            pl.BlockSpec((1, FBLK, D), lambda e, f, t: (e, f, 0)),
            pl.BlockSpec((E, D), lambda e, f, t: (0, 0)),
        ],
        out_specs=pl.BlockSpec((NTOK, D), lambda e, f, t: (0, 0)),
        out_shape=jax.ShapeDtypeStruct((NTOK, D), jnp.float32),
        scratch_shapes=[pltpu.VMEM((NTOK, E), jnp.float32)],
    )(x, Wg, bg.reshape(1, E), W1, b1, W2, b2)
    return out


# speedup vs baseline: 1.0152x; 1.0152x over previous
import functools

import jax
import jax.numpy as jnp
from jax import lax
from jax.experimental import pallas as pl
from jax.experimental.pallas import tpu as pltpu
from jax.experimental.pallas import tpu_sc as plsc

D = 768
FF = 3072
E = 8
NTOK = 2048
K = 2
PAIRS = NTOK * K
TM = 256
NTILES = PAIRS // TM + E
P = NTILES * TM
NT_PAD = 32

NC = 2
NS = 16
NW = NC * NS
_NEG = -1e30


def _route_body(x_ref, Wg_ref, bg_ref, pos_ref, w_ref, sched_ref,
                c_ref, excl_ref):
    s = jnp.dot(x_ref[...], Wg_ref[...],
                preferred_element_type=jnp.float32) + bg_ref[...]
    col = lax.broadcasted_iota(jnp.int32, s.shape, 1)
    m1 = jnp.max(s, axis=1, keepdims=True)
    a1 = jnp.min(jnp.where(s == m1, col, E), axis=1, keepdims=True)
    s2 = jnp.where(col == a1, _NEG, s)
    m2 = jnp.max(s2, axis=1, keepdims=True)
    a2 = jnp.min(jnp.where(s2 == m2, col, E), axis=1, keepdims=True)
    e2 = jnp.exp(m2 - m1)
    w1v = 1.0 / (1.0 + e2)
    w2v = e2 / (1.0 + e2)
    oh1 = (col == a1).astype(jnp.float32)
    oh2 = (col == a2).astype(jnp.float32)
    c_ref[...] = oh1 + oh2

    r = lax.broadcasted_iota(jnp.int32, (TM, TM), 0)
    c2 = lax.broadcasted_iota(jnp.int32, (TM, TM), 1)
    ltri = (r > c2).astype(jnp.float32)

    def chunk(i, carry):
        cc = c_ref[pl.ds(i * TM, TM), :]
        excl_ref[pl.ds(i * TM, TM), :] = (
            jnp.dot(ltri, cc, preferred_element_type=jnp.float32) + carry)
        return carry + jnp.sum(cc, axis=0, keepdims=True)

    counts = lax.fori_loop(0, NTOK // TM, chunk,
                           jnp.zeros((1, E), jnp.float32))

    tiles_e = jnp.ceil(counts / TM)
    eu = lax.broadcasted_iota(jnp.int32, (E, E), 0)
    ev = lax.broadcasted_iota(jnp.int32, (E, E), 1)
    utri = (eu < ev).astype(jnp.float32)
    tile_start = jnp.dot(tiles_e, utri,
                         preferred_element_type=jnp.float32)
    off_pad = tile_start * TM

    excl = excl_ref[...]
    rank1 = jnp.sum(oh1 * excl, axis=1, keepdims=True)
    rank2 = jnp.sum(oh2 * excl, axis=1, keepdims=True)
    base1 = jnp.sum(oh1 * off_pad, axis=1, keepdims=True)
    base2 = jnp.sum(oh2 * off_pad, axis=1, keepdims=True)
    pos_ref[...] = jnp.concatenate(
        [base1 + rank1, base2 + rank2], axis=1).astype(jnp.int32)
    w_ref[...] = jnp.concatenate([w1v, w2v], axis=1)

    tend = (tile_start + tiles_e).astype(jnp.int32)
    trow = lax.broadcasted_iota(jnp.int32, (NT_PAD, E), 0)
    m = (trow >= jnp.broadcast_to(tend, (NT_PAD, E))).astype(jnp.float32)
    texp = jnp.sum(m, axis=1, keepdims=True)
    sched_ref[...] = jnp.broadcast_to(texp, (NT_PAD, 128)).astype(jnp.int32)


def _route(x, Wg, bg):
    return pl.pallas_call(
        _route_body,
        out_shape=(
            jax.ShapeDtypeStruct((NTOK, K), jnp.int32),
            jax.ShapeDtypeStruct((NTOK, K), jnp.float32),
            jax.ShapeDtypeStruct((NT_PAD, 128), jnp.int32),
        ),
        scratch_shapes=[
            pltpu.VMEM((NTOK, E), jnp.float32),
            pltpu.VMEM((NTOK, E), jnp.float32),
        ],
    )(x, Wg, bg.reshape(1, E))


_CH = PAIRS // NW


def _scatter_kernel():
    return functools.partial(
        pl.kernel,
        out_type=(jax.ShapeDtypeStruct((P,), jnp.int32),
                  jax.ShapeDtypeStruct((P,), jnp.float32)),
        mesh=plsc.VectorSubcoreMesh(core_axis_name="c", subcore_axis_name="s"),
        scratch_types=[
            pltpu.VMEM((_CH,), jnp.int32),
            pltpu.VMEM((_CH,), jnp.float32),
            pltpu.VMEM((_CH,), jnp.int32),
            pltpu.SemaphoreType.DMA,
        ],
    )(_scatter_body)


def _scatter_body(pos_hbm, w_hbm, tok_out, w_out, idx_v, wv, tokv, sem):
    wid = lax.axis_index("s") * NC + lax.axis_index("c")
    base = wid * _CH
    pltpu.sync_copy(pos_hbm.at[pl.ds(base, _CH)], idx_v)
    pltpu.sync_copy(w_hbm.at[pl.ds(base, _CH)], wv)
    for j in range(_CH // 16):
        pv = lax.iota(jnp.int32, 16) + (base + j * 16)
        tokv[pl.ds(j * 16, 16)] = lax.shift_right_logical(pv, 1)
    pltpu.async_copy(tokv, tok_out.at[idx_v], sem).wait()
    pltpu.async_copy(wv, w_out.at[idx_v], sem).wait()


def _sc_scatter(pos_flat, w_flat):
    return _scatter_kernel()(pos_flat, w_flat)


_RPW = P // NW
_GCH = _RPW // 2


def _gather_kernel():
    return functools.partial(
        pl.kernel,
        out_type=jax.ShapeDtypeStruct((P, D), jnp.float32),
        mesh=plsc.VectorSubcoreMesh(core_axis_name="c", subcore_axis_name="s"),
        scratch_types=[
            pltpu.VMEM((_GCH,), jnp.int32),
            pltpu.VMEM((_GCH, D), jnp.float32),
            pltpu.SemaphoreType.DMA,
        ],
    )(_gather_body)


def _gather_body(x_hbm, tok_hbm, xs_out, idx_v, rows_v, sem):
    wid = lax.axis_index("s") * NC + lax.axis_index("c")
    for c in range(2):
        base = wid * _RPW + c * _GCH
        pltpu.sync_copy(tok_hbm.at[pl.ds(base, _GCH)], idx_v)
        for j in range(_GCH // 16):
            idx_v[pl.ds(j * 16, 16)] = lax.bitwise_and(
                idx_v[pl.ds(j * 16, 16)], NTOK - 1)
        pltpu.async_copy(x_hbm.at[idx_v], rows_v, sem).wait()
        pltpu.sync_copy(rows_v, xs_out.at[pl.ds(base, _GCH)])


def _sc_gather(x, src_tok):
    return _gather_kernel()(x, src_tok)


def _ffn_body(s_ref, x_ref, W1_ref, b1_ref, W2_ref, b2_ref, wr_ref, out_ref):
    t = pl.program_id(0)
    e = s_ref[t]

    @pl.when(e < E)
    def _():
        h = jnp.maximum(
            jnp.dot(x_ref[...], W1_ref[0],
                    preferred_element_type=jnp.float32)
            + b1_ref[pl.ds(e, 1), :], 0.0)
        y = (jnp.dot(h, W2_ref[0], preferred_element_type=jnp.float32)
             + b2_ref[pl.ds(e, 1), :])
        out_ref[...] = y * wr_ref[...]


def _ffn(xs, W1, b1, W2, b2, w_sorted, sched):
    def clamp(v):
        return jnp.minimum(v, E - 1)

    grid_spec = pltpu.PrefetchScalarGridSpec(
        num_scalar_prefetch=1,
        grid=(NTILES,),
        in_specs=[
            pl.BlockSpec((TM, D), lambda t, s: (t, 0)),
            pl.BlockSpec((1, D, FF), lambda t, s: (clamp(s[t]), 0, 0)),
            pl.BlockSpec((E, FF), lambda t, s: (0, 0)),
            pl.BlockSpec((1, FF, D), lambda t, s: (clamp(s[t]), 0, 0)),
            pl.BlockSpec((E, D), lambda t, s: (0, 0)),
            pl.BlockSpec((TM, 1), lambda t, s: (t, 0)),
        ],
        out_specs=pl.BlockSpec((TM, D), lambda t, s: (t, 0)),
    )
    return pl.pallas_call(
        _ffn_body,
        grid_spec=grid_spec,
        out_shape=jax.ShapeDtypeStruct((P, D), jnp.float32),
    )(sched, xs, W1, b1, W2, b2, w_sorted.reshape(P, 1))


_TPW = NTOK // NW
_TCH = _TPW // 2


def _combine_kernel():
    return functools.partial(
        pl.kernel,
        out_type=jax.ShapeDtypeStruct((NTOK, D), jnp.float32),
        mesh=plsc.VectorSubcoreMesh(core_axis_name="c", subcore_axis_name="s"),
        scratch_types=[
            pltpu.VMEM((2 * _TCH,), jnp.int32),
            pltpu.VMEM((2 * _TCH, D), jnp.float32),
            pltpu.VMEM((_TCH, D), jnp.float32),
            pltpu.SemaphoreType.DMA,
        ],
    )(_combine_body)


def _combine_body(ys_hbm, pos_hbm, out_hbm, idx_v, rows_v, o_v, sem):
    wid = lax.axis_index("s") * NC + lax.axis_index("c")
    for c in range(2):
        tbase = wid * _TPW + c * _TCH
        pltpu.sync_copy(pos_hbm.at[pl.ds(tbase * K, K * _TCH)], idx_v)
        pltpu.async_copy(ys_hbm.at[idx_v], rows_v, sem).wait()

        def tok(j, _):
            for q in range(D // 16):
                sl = pl.ds(q * 16, 16)
                o_v[j, sl] = rows_v[2 * j, sl] + rows_v[2 * j + 1, sl]
            return 0

        lax.fori_loop(0, _TCH, tok, 0)
        pltpu.sync_copy(o_v, out_hbm.at[pl.ds(tbase, _TCH)])


def _sc_combine(ys, pos_flat):
    return _combine_kernel()(ys, pos_flat)


def kernel(x, Wg, bg, W1, b1, W2, b2):
    pos2, w2d, sched2d = _route(x, Wg, bg)
    sched = sched2d[:, 0]
    pos_flat = pos2.reshape(PAIRS)
    w_flat = w2d.reshape(PAIRS)
    src_tok, w_sorted = _sc_scatter(pos_flat, w_flat)
    xs = _sc_gather(x, src_tok)
    ys = _ffn(xs, W1, b1, W2, b2, w_sorted, sched)
    out = _sc_combine(ys, pos_flat)
    return out


# speedup vs baseline: 1.8026x; 1.7757x over previous
import functools

import jax
import jax.numpy as jnp
from jax import lax
from jax.experimental import pallas as pl
from jax.experimental.pallas import tpu as pltpu
from jax.experimental.pallas import tpu_sc as plsc

D = 768
FF = 3072
E = 8
NTOK = 2048
K = 2
PAIRS = NTOK * K
TM = 256
NTILES = PAIRS // TM + E
P = NTILES * TM
NT_PAD = 32

NC = 2
NS = 16
NW = NC * NS
_NEG = -1e30


def _route_body(x_ref, Wg_ref, bg_ref, pos_ref, w_ref, sched_ref,
                c_ref, excl_ref):
    s = jnp.dot(x_ref[...], Wg_ref[...],
                preferred_element_type=jnp.float32) + bg_ref[...]
    col = lax.broadcasted_iota(jnp.int32, s.shape, 1)
    m1 = jnp.max(s, axis=1, keepdims=True)
    a1 = jnp.min(jnp.where(s == m1, col, E), axis=1, keepdims=True)
    s2 = jnp.where(col == a1, _NEG, s)
    m2 = jnp.max(s2, axis=1, keepdims=True)
    a2 = jnp.min(jnp.where(s2 == m2, col, E), axis=1, keepdims=True)
    e2 = jnp.exp(m2 - m1)
    w1v = 1.0 / (1.0 + e2)
    w2v = e2 / (1.0 + e2)
    oh1 = (col == a1).astype(jnp.float32)
    oh2 = (col == a2).astype(jnp.float32)
    c_ref[...] = oh1 + oh2

    r = lax.broadcasted_iota(jnp.int32, (TM, TM), 0)
    c2 = lax.broadcasted_iota(jnp.int32, (TM, TM), 1)
    ltri = (r > c2).astype(jnp.float32)

    def chunk(i, carry):
        cc = c_ref[pl.ds(i * TM, TM), :]
        excl_ref[pl.ds(i * TM, TM), :] = (
            jnp.dot(ltri, cc, preferred_element_type=jnp.float32) + carry)
        return carry + jnp.sum(cc, axis=0, keepdims=True)

    counts = lax.fori_loop(0, NTOK // TM, chunk,
                           jnp.zeros((1, E), jnp.float32))

    tiles_e = jnp.ceil(counts / TM)
    eu = lax.broadcasted_iota(jnp.int32, (E, E), 0)
    ev = lax.broadcasted_iota(jnp.int32, (E, E), 1)
    utri = (eu < ev).astype(jnp.float32)
    tile_start = jnp.dot(tiles_e, utri,
                         preferred_element_type=jnp.float32)
    off_pad = tile_start * TM

    excl = excl_ref[...]
    rank1 = jnp.sum(oh1 * excl, axis=1, keepdims=True)
    rank2 = jnp.sum(oh2 * excl, axis=1, keepdims=True)
    base1 = jnp.sum(oh1 * off_pad, axis=1, keepdims=True)
    base2 = jnp.sum(oh2 * off_pad, axis=1, keepdims=True)
    pos_ref[...] = jnp.concatenate(
        [base1 + rank1, base2 + rank2], axis=1).astype(jnp.int32)
    w_ref[...] = jnp.concatenate([w1v, w2v], axis=1)

    tend = (tile_start + tiles_e).astype(jnp.int32)
    trow = lax.broadcasted_iota(jnp.int32, (NT_PAD, E), 0)
    m = (trow >= jnp.broadcast_to(tend, (NT_PAD, E))).astype(jnp.float32)
    texp = jnp.sum(m, axis=1, keepdims=True)
    sched_ref[...] = jnp.broadcast_to(texp, (NT_PAD, 128)).astype(jnp.int32)


def _route(x, Wg, bg):
    return pl.pallas_call(
        _route_body,
        out_shape=(
            jax.ShapeDtypeStruct((NTOK, K), jnp.int32),
            jax.ShapeDtypeStruct((NTOK, K), jnp.float32),
            jax.ShapeDtypeStruct((NT_PAD, 128), jnp.int32),
        ),
        scratch_shapes=[
            pltpu.VMEM((NTOK, E), jnp.float32),
            pltpu.VMEM((NTOK, E), jnp.float32),
        ],
    )(x, Wg, bg.reshape(1, E))


def _ffn_body(s_ref, x_ref, pos_ref, w_ref, W1_ref, b1_ref, W2_ref, b2_ref,
              out_ref):
    t = pl.program_id(0)
    e = s_ref[t]

    @pl.when(e < E)
    def _():
        li = lax.broadcasted_iota(jnp.int32, (NTOK, TM), 1) + t * TM
        a0 = (pos_ref[:, 0:1] == li).astype(jnp.float32)
        a1 = (pos_ref[:, 1:2] == li).astype(jnp.float32)
        dn = (((0,), (0,)), ((), ()))
        xt = lax.dot_general(a0 + a1, x_ref[...], dn,
                             preferred_element_type=jnp.float32)
        wt = (lax.dot_general(a0, w_ref[:, 0:1], dn,
                              preferred_element_type=jnp.float32)
              + lax.dot_general(a1, w_ref[:, 1:2], dn,
                                preferred_element_type=jnp.float32))
        h = jnp.maximum(
            jnp.dot(xt, W1_ref[0], preferred_element_type=jnp.float32)
            + b1_ref[pl.ds(e, 1), :], 0.0)
        y = (jnp.dot(h, W2_ref[0], preferred_element_type=jnp.float32)
             + b2_ref[pl.ds(e, 1), :])
        out_ref[...] = y * wt


def _ffn(x, pos2, w2d, W1, b1, W2, b2, sched):
    def clamp(v):
        return jnp.minimum(v, E - 1)

    grid_spec = pltpu.PrefetchScalarGridSpec(
        num_scalar_prefetch=1,
        grid=(NTILES,),
        in_specs=[
            pl.BlockSpec((NTOK, D), lambda t, s: (0, 0)),
            pl.BlockSpec((NTOK, K), lambda t, s: (0, 0)),
            pl.BlockSpec((NTOK, K), lambda t, s: (0, 0)),
            pl.BlockSpec((1, D, FF), lambda t, s: (clamp(s[t]), 0, 0)),
            pl.BlockSpec((E, FF), lambda t, s: (0, 0)),
            pl.BlockSpec((1, FF, D), lambda t, s: (clamp(s[t]), 0, 0)),
            pl.BlockSpec((E, D), lambda t, s: (0, 0)),
        ],
        out_specs=pl.BlockSpec((TM, D), lambda t, s: (t, 0)),
    )
    return pl.pallas_call(
        _ffn_body,
        grid_spec=grid_spec,
        out_shape=jax.ShapeDtypeStruct((P, D), jnp.float32),
    )(sched, x, pos2, w2d, W1, b1, W2, b2)


_TPW = NTOK // NW
_TCH = _TPW // 2


def _combine_kernel():
    return functools.partial(
        pl.kernel,
        out_type=jax.ShapeDtypeStruct((NTOK, D), jnp.float32),
        mesh=plsc.VectorSubcoreMesh(core_axis_name="c", subcore_axis_name="s"),
        scratch_types=[
            pltpu.VMEM((2 * _TCH,), jnp.int32),
            pltpu.VMEM((2 * _TCH, D), jnp.float32),
            pltpu.VMEM((_TCH, D), jnp.float32),
            pltpu.SemaphoreType.DMA,
        ],
    )(_combine_body)


def _combine_body(ys_hbm, pos_hbm, out_hbm, idx_v, rows_v, o_v, sem):
    wid = lax.axis_index("s") * NC + lax.axis_index("c")
    for c in range(2):
        tbase = wid * _TPW + c * _TCH
        pltpu.sync_copy(pos_hbm.at[pl.ds(tbase * K, K * _TCH)], idx_v)
        pltpu.async_copy(ys_hbm.at[idx_v], rows_v, sem).wait()

        def tok(j, _):
            for q in range(D // 16):
                sl = pl.ds(q * 16, 16)
                o_v[j, sl] = rows_v[2 * j, sl] + rows_v[2 * j + 1, sl]
            return 0

        lax.fori_loop(0, _TCH, tok, 0)
        pltpu.sync_copy(o_v, out_hbm.at[pl.ds(tbase, _TCH)])


def _sc_combine(ys, pos_flat):
    return _combine_kernel()(ys, pos_flat)


def kernel(x, Wg, bg, W1, b1, W2, b2):
    pos2, w2d, sched2d = _route(x, Wg, bg)
    sched = sched2d[:, 0]
    ys = _ffn(x, pos2, w2d, W1, b1, W2, b2, sched)
    out = _sc_combine(ys, pos2.reshape(PAIRS))
    return out
